# baseline (speedup 1.0000x reference)
import functools

import jax
import jax.numpy as jnp
from jax import lax
from jax.experimental import pallas as pl
from jax.experimental.pallas import tpu as pltpu
from jax.experimental.pallas import tpu_sc as plsc

_NC = 2
_NS = 16
_L = 16
_R = 5
_PP = 4
_FOLD = 4


def _make_sc_kernel(n_pairs, n_rel, max_rel, emb, n_out):
    n_workers = _NC * _NS
    assert n_pairs % n_workers == 0 and n_out % _R == 0
    pairs_per_w = n_pairs // n_workers
    blocks_per_pair = n_out // _R
    n_blocks = pairs_per_w * blocks_per_pair
    assert n_blocks % 2 == 0

    n_full = emb // _L
    offs = [c * _L for c in range(n_full)]
    tail = emb % _L
    emb_main = n_full * _L

    mesh = plsc.VectorSubcoreMesh(core_axis_name="c", subcore_axis_name="s")

    @functools.partial(
        pl.kernel,
        mesh=mesh,
        out_type=(
            jax.ShapeDtypeStruct((n_pairs, n_out, emb_main), jnp.float32),
            jax.ShapeDtypeStruct((n_pairs, n_out, _L), jnp.float32),
        ),
        scratch_types=[
            pltpu.VMEM((_R, max_rel, emb), jnp.float32),
            pltpu.VMEM((_R, max_rel, emb), jnp.float32),
            pltpu.VMEM((n_out, emb_main), jnp.float32),
            pltpu.VMEM((n_out, _L), jnp.float32),
            pltpu.VMEM((n_pairs, _L), jnp.int32),
            pltpu.SemaphoreType.DMA,
            pltpu.SemaphoreType.DMA,
        ],
    )
    def sc_kernel(adj_hbm, lim_hbm, out_hbm, tout_hbm, buf_a, buf_b, obuf,
                  tbuf, limv, sem_a, sem_b):
        w = lax.axis_index("s") * _NC + lax.axis_index("c")
        p_base = w * pairs_per_w

        pltpu.sync_copy(lim_hbm, limv)

        def blk_src(blk):
            p = p_base + blk // blocks_per_pair
            j0 = (blk % blocks_per_pair) * _R
            return adj_hbm.at[p, pl.ds(j0, _R)]

        def start(blk, buf, sem):
            pltpu.async_copy(blk_src(blk), buf, sem)

        def wait(blk, buf, sem):
            pltpu.make_async_copy(blk_src(blk), buf, sem).wait()

        def compute(blk, buf):
            p = p_base + blk // blocks_per_pair
            jl0 = (blk % blocks_per_pair) * _R
            limvec = limv[p]

            def tree_sum(jr, off):
                vals = [buf[jr, k, pl.ds(off, _L)] for k in range(max_rel)]
                while len(vals) > 1:
                    nxt = [vals[i] + vals[i + 1]
                           for i in range(0, len(vals) - 1, 2)]
                    if len(vals) % 2:
                        nxt.append(vals[-1])
                    vals = nxt
                return vals[0]

            def row(jr, _):
                keep = jnp.broadcast_to(jl0 + jr, (_L,)) < limvec
                for off in offs:
                    v = tree_sum(jr, off)
                    obuf[jl0 + jr, pl.ds(off, _L)] = jnp.where(keep, v, 0.0)
                if tail:
                    v = tree_sum(jr, emb - _L)
                    tbuf[jl0 + jr, pl.ds(0, _L)] = jnp.where(keep, v, 0.0)
                return 0

            lax.fori_loop(0, _R, row, 0, unroll=True)

        start(0, buf_a, sem_a)

        def step(bp, _):
            blk_a = 2 * bp
            blk_b = 2 * bp + 1
            start(blk_b, buf_b, sem_b)
            wait(blk_a, buf_a, sem_a)
            compute(blk_a, buf_a)

            @pl.when(bp < n_blocks // 2 - 1)
            def _():
                start(blk_a + 2, buf_a, sem_a)

            wait(blk_b, buf_b, sem_b)
            compute(blk_b, buf_b)

            @pl.when(lax.rem(bp, blocks_per_pair // 2)
                     == blocks_per_pair // 2 - 1)
            def _():
                p = p_base + bp // (blocks_per_pair // 2)
                pltpu.sync_copy(obuf, out_hbm.at[p])
                if tail:
                    pltpu.sync_copy(tbuf, tout_hbm.at[p])

            return 0

        lax.fori_loop(0, n_blocks // 2, step, 0)

    return sc_kernel


def _make_tc_kernel(n_pairs, n_sc, n_rel, max_rel, emb, n_out):
    n_tc = n_pairs - n_sc
    pp = _PP
    assert n_tc % pp == 0 and n_sc % pp == 0
    wide = _FOLD * emb
    rows = n_out * max_rel // _FOLD

    def body(lim_ref, m_ref, adj_ref, out_ref):
        g = pl.program_id(0)
        for q in range(pp):
            x = adj_ref[q]
            z = jnp.dot(m_ref[...], x, preferred_element_type=jnp.float32)
            s = z[:, :emb]
            for t in range(1, _FOLD):
                s = s + z[:, t * emb:(t + 1) * emb]
            limv = lim_ref[g * pp + q + n_sc]
            mask = lax.broadcasted_iota(jnp.int32, (n_out, emb), 0) < limv
            out_ref[0, q] = jnp.where(mask, s, 0.0)

    return pl.pallas_call(
        body,
        grid_spec=pltpu.PrefetchScalarGridSpec(
            num_scalar_prefetch=1,
            grid=(n_tc // pp,),
            in_specs=[
                pl.BlockSpec((n_out, rows), lambda g, lim: (0, 0)),
                pl.BlockSpec((pp, rows, wide),
                             lambda g, lim: (g + n_sc // pp, 0, 0)),
            ],
            out_specs=pl.BlockSpec((1, pp, n_out, emb),
                                   lambda g, lim: (g + n_sc // pp, 0, 0, 0)),
        ),
        out_shape=jax.ShapeDtypeStruct((n_pairs // pp, pp, n_out, emb),
                                       jnp.float32),
    )


def kernel(adj_list, original_limit, batch_size, max_original_nodes=30,
           keep_original=True):
    b, n_rounds, n_rel, max_rel, emb = adj_list.shape
    n_out = 30
    n_pairs = b * n_rounds
    n_sc = 32

    adj4 = adj_list.reshape(n_pairs, n_rel, max_rel, emb)

    lim = original_limit.reshape(n_pairs).astype(jnp.int32)
    mon = jnp.asarray(max_original_nodes, jnp.int32)
    eff = jnp.where(jnp.asarray(keep_original, jnp.bool_),
                    jnp.minimum(lim, mon),
                    jnp.broadcast_to(mon, lim.shape))
    eff = jnp.clip(eff, 0, n_out)
    lim_splat = jnp.broadcast_to(eff[:n_sc, None], (n_sc, _L))

    sck = _make_sc_kernel(n_sc, n_rel, max_rel, emb, n_out)
    out_main, out_tail = sck(adj4, lim_splat)

    rpj = max_rel // _FOLD
    rows = n_out * rpj
    sel = (lax.broadcasted_iota(jnp.int32, (n_out, rows), 1)
           // rpj == lax.broadcasted_iota(jnp.int32, (n_out, rows), 0)
           ).astype(jnp.float32)
    adj3 = adj_list.reshape(n_pairs, n_rel * rpj, _FOLD * emb)
    tck = _make_tc_kernel(n_pairs, n_sc, n_rel, max_rel, emb, n_out)
    out_all = tck(eff, sel, adj3).reshape(n_pairs, n_out, emb)

    tail = emb % _L
    if tail:
        out_sc = jnp.concatenate(
            [out_main, out_tail[:, :, _L - tail:]], axis=2)
    else:
        out_sc = out_main
    out = lax.dynamic_update_slice(out_all, out_sc, (0, 0, 0))
    return out.reshape(b, n_rounds, n_out, emb)


# speedup vs baseline: 4.2228x; 4.2228x over previous
import functools

import jax
import jax.numpy as jnp
from jax import lax
from jax.experimental import pallas as pl
from jax.experimental.pallas import tpu as pltpu
from jax.experimental.pallas import tpu_sc as plsc

_NC = 2
_NS = 16
_L = 16
_R = 5
_PP = 4
_FOLD = 1


def _make_sc_kernel(n_pairs, n_rel, max_rel, emb, n_out):
    n_workers = _NC * _NS
    assert n_pairs % n_workers == 0 and n_out % _R == 0
    pairs_per_w = n_pairs // n_workers
    blocks_per_pair = n_out // _R
    n_blocks = pairs_per_w * blocks_per_pair
    assert n_blocks % 2 == 0

    n_full = emb // _L
    offs = [c * _L for c in range(n_full)]
    tail = emb % _L
    emb_main = n_full * _L

    mesh = plsc.VectorSubcoreMesh(core_axis_name="c", subcore_axis_name="s")

    @functools.partial(
        pl.kernel,
        mesh=mesh,
        out_type=(
            jax.ShapeDtypeStruct((n_pairs, n_out, emb_main), jnp.float32),
            jax.ShapeDtypeStruct((n_pairs, n_out, _L), jnp.float32),
        ),
        scratch_types=[
            pltpu.VMEM((_R, max_rel, emb), jnp.float32),
            pltpu.VMEM((_R, max_rel, emb), jnp.float32),
            pltpu.VMEM((n_out, emb_main), jnp.float32),
            pltpu.VMEM((n_out, _L), jnp.float32),
            pltpu.VMEM((n_pairs, _L), jnp.int32),
            pltpu.SemaphoreType.DMA,
            pltpu.SemaphoreType.DMA,
        ],
    )
    def sc_kernel(adj_hbm, lim_hbm, out_hbm, tout_hbm, buf_a, buf_b, obuf,
                  tbuf, limv, sem_a, sem_b):
        w = lax.axis_index("s") * _NC + lax.axis_index("c")
        p_base = w * pairs_per_w

        pltpu.sync_copy(lim_hbm, limv)

        def blk_src(blk):
            p = p_base + blk // blocks_per_pair
            j0 = (blk % blocks_per_pair) * _R
            return adj_hbm.at[p, pl.ds(j0, _R)]

        def start(blk, buf, sem):
            pltpu.async_copy(blk_src(blk), buf, sem)

        def wait(blk, buf, sem):
            pltpu.make_async_copy(blk_src(blk), buf, sem).wait()

        def compute(blk, buf):
            p = p_base + blk // blocks_per_pair
            jl0 = (blk % blocks_per_pair) * _R
            limvec = limv[p]

            def tree_sum(jr, off):
                vals = [buf[jr, k, pl.ds(off, _L)] for k in range(max_rel)]
                while len(vals) > 1:
                    nxt = [vals[i] + vals[i + 1]
                           for i in range(0, len(vals) - 1, 2)]
                    if len(vals) % 2:
                        nxt.append(vals[-1])
                    vals = nxt
                return vals[0]

            def row(jr, _):
                keep = jnp.broadcast_to(jl0 + jr, (_L,)) < limvec
                for off in offs:
                    v = tree_sum(jr, off)
                    obuf[jl0 + jr, pl.ds(off, _L)] = jnp.where(keep, v, 0.0)
                if tail:
                    v = tree_sum(jr, emb - _L)
                    tbuf[jl0 + jr, pl.ds(0, _L)] = jnp.where(keep, v, 0.0)
                return 0

            lax.fori_loop(0, _R, row, 0, unroll=True)

        start(0, buf_a, sem_a)

        def step(bp, _):
            blk_a = 2 * bp
            blk_b = 2 * bp + 1
            start(blk_b, buf_b, sem_b)
            wait(blk_a, buf_a, sem_a)
            compute(blk_a, buf_a)

            @pl.when(bp < n_blocks // 2 - 1)
            def _():
                start(blk_a + 2, buf_a, sem_a)

            wait(blk_b, buf_b, sem_b)
            compute(blk_b, buf_b)

            @pl.when(lax.rem(bp, blocks_per_pair // 2)
                     == blocks_per_pair // 2 - 1)
            def _():
                p = p_base + bp // (blocks_per_pair // 2)
                pltpu.sync_copy(obuf, out_hbm.at[p])
                if tail:
                    pltpu.sync_copy(tbuf, tout_hbm.at[p])

            return 0

        lax.fori_loop(0, n_blocks // 2, step, 0)

    return sc_kernel


def _make_tc_kernel(n_pairs, n_sc, n_rel, max_rel, emb, n_out):
    n_tc = n_pairs - n_sc
    pp = _PP
    assert n_tc % pp == 0 and n_sc % pp == 0
    wide = _FOLD * emb
    rows = n_out * max_rel // _FOLD

    def body(lim_ref, m_ref, adj_ref, out_ref):
        g = pl.program_id(0)
        for q in range(pp):
            x = adj_ref[q]
            z = jnp.dot(m_ref[...], x, preferred_element_type=jnp.float32)
            s = z[:, :emb]
            for t in range(1, _FOLD):
                s = s + z[:, t * emb:(t + 1) * emb]
            limv = lim_ref[g * pp + q + n_sc]
            mask = lax.broadcasted_iota(jnp.int32, (n_out, emb), 0) < limv
            out_ref[0, q] = jnp.where(mask, s, 0.0)

    return pl.pallas_call(
        body,
        grid_spec=pltpu.PrefetchScalarGridSpec(
            num_scalar_prefetch=1,
            grid=(n_tc // pp,),
            in_specs=[
                pl.BlockSpec((n_out, rows), lambda g, lim: (0, 0)),
                pl.BlockSpec((pp, rows, wide),
                             lambda g, lim: (g + n_sc // pp, 0, 0)),
            ],
            out_specs=pl.BlockSpec((1, pp, n_out, emb),
                                   lambda g, lim: (g + n_sc // pp, 0, 0, 0)),
        ),
        out_shape=jax.ShapeDtypeStruct((n_pairs // pp, pp, n_out, emb),
                                       jnp.float32),
    )


def kernel(adj_list, original_limit, batch_size, max_original_nodes=30,
           keep_original=True):
    b, n_rounds, n_rel, max_rel, emb = adj_list.shape
    n_out = 30
    n_pairs = b * n_rounds
    n_sc = 32

    adj4 = adj_list.reshape(n_pairs, n_rel, max_rel, emb)

    lim = original_limit.reshape(n_pairs).astype(jnp.int32)
    mon = jnp.asarray(max_original_nodes, jnp.int32)
    eff = jnp.where(jnp.asarray(keep_original, jnp.bool_),
                    jnp.minimum(lim, mon),
                    jnp.broadcast_to(mon, lim.shape))
    eff = jnp.clip(eff, 0, n_out)
    lim_splat = jnp.broadcast_to(eff[:n_sc, None], (n_sc, _L))

    sck = _make_sc_kernel(n_sc, n_rel, max_rel, emb, n_out)
    out_main, out_tail = sck(adj4, lim_splat)

    rpj = max_rel // _FOLD
    rows = n_out * rpj
    sel = (lax.broadcasted_iota(jnp.int32, (n_out, rows), 1)
           // rpj == lax.broadcasted_iota(jnp.int32, (n_out, rows), 0)
           ).astype(jnp.float32)
    adj3 = adj_list.reshape(n_pairs, n_rel * rpj, _FOLD * emb)
    tck = _make_tc_kernel(n_pairs, n_sc, n_rel, max_rel, emb, n_out)
    out_all = tck(eff, sel, adj3).reshape(n_pairs, n_out, emb)

    tail = emb % _L
    if tail:
        out_sc = jnp.concatenate(
            [out_main, out_tail[:, :, _L - tail:]], axis=2)
    else:
        out_sc = out_main
    out = lax.dynamic_update_slice(out_all, out_sc, (0, 0, 0))
    return out.reshape(b, n_rounds, n_out, emb)
